# baseline (speedup 1.0000x reference)
import functools

import jax
import jax.numpy as jnp
from jax import lax
from jax.experimental import pallas as pl
from jax.experimental.pallas import tpu as pltpu
from jax.experimental.pallas import tpu_sc as plsc

DIM = 1024
DFF = 4096
E = 8
K = 2
BM = 128
BDFF = 1024

_INTERPRET = False


def _router_body(x_ref, wg_ref, vals_ref, pos_ref, be_ref, aux_ref):
    x = x_ref[...]
    s = x.shape[0]
    logits = jax.lax.dot_general(
        x, wg_ref[...], (((1,), (1,)), ((), ())),
        preferred_element_type=jnp.float32,
        precision=jax.lax.Precision.DEFAULT)
    m = jnp.max(logits, axis=-1, keepdims=True)
    ex = jnp.exp(logits - m)
    p = ex / jnp.sum(ex, axis=-1, keepdims=True)
    iota = jax.lax.broadcasted_iota(jnp.int32, p.shape, 1)
    v0 = jnp.max(p, axis=-1, keepdims=True)
    i0 = jnp.min(jnp.where(p == v0, iota, E), axis=-1, keepdims=True)
    oh0 = (iota == i0).astype(jnp.float32)
    p1 = jnp.where(oh0 > 0, -1.0, p)
    v1 = jnp.max(p1, axis=-1, keepdims=True)
    i1 = jnp.min(jnp.where(p1 == v1, iota, E), axis=-1, keepdims=True)
    oh1 = (iota == i1).astype(jnp.float32)
    vals_ref[...] = jnp.concatenate([v0, v1], axis=1)

    oh = oh0 + oh1
    cs = 256
    tri = (jax.lax.broadcasted_iota(jnp.int32, (cs, cs), 0)
           > jax.lax.broadcasted_iota(jnp.int32, (cs, cs), 1)
           ).astype(jnp.float32)
    chunks = []
    carry = jnp.zeros((1, E), jnp.float32)
    for c in range(s // cs):
        blk = oh[c * cs:(c + 1) * cs]
        local = jax.lax.dot_general(
            tri, blk, (((1,), (0,)), ((), ())),
            preferred_element_type=jnp.float32)
        chunks.append(local + carry)
        carry = carry + jnp.sum(blk, axis=0, keepdims=True)
    prev = jnp.concatenate(chunks, axis=0)
    counts = carry

    nblk = jnp.floor((counts + float(BM - 1)) * (1.0 / BM))
    tri8 = (jax.lax.broadcasted_iota(jnp.int32, (E, E), 0)
            <= jax.lax.broadcasted_iota(jnp.int32, (E, E), 1)
            ).astype(jnp.float32)
    ends = jax.lax.dot_general(
        nblk, tri8, (((1,), (0,)), ((), ())),
        preferred_element_type=jnp.float32)
    offs = (ends - nblk) * float(BM)
    rank0 = jnp.sum(prev * oh0, axis=1, keepdims=True)
    rank1 = jnp.sum(prev * oh1, axis=1, keepdims=True)
    off0 = jnp.sum(offs * oh0, axis=1, keepdims=True)
    off1 = jnp.sum(offs * oh1, axis=1, keepdims=True)
    pos_ref[...] = jnp.concatenate(
        [off0 + rank0, off1 + rank1], axis=1).astype(jnp.int32)

    nb = be_ref.shape[1]
    biota = jax.lax.broadcasted_iota(jnp.int32, (E, nb), 1).astype(jnp.float32)
    be = jnp.sum((biota >= ends.reshape(E, 1)).astype(jnp.int32),
                 axis=0, keepdims=True)
    be_ref[...] = jnp.minimum(be, E - 1)

    imp = jnp.sum(p, axis=0, keepdims=True)
    aux_ref[...] = (E / (float(s) * float(s))) * jnp.sum(
        imp * counts).reshape(1, 1)


def _router(x, Wg, nb):
    s = x.shape[0]
    return pl.pallas_call(
        _router_body,
        out_shape=(
            jax.ShapeDtypeStruct((s, K), jnp.float32),
            jax.ShapeDtypeStruct((s, K), jnp.int32),
            jax.ShapeDtypeStruct((1, nb), jnp.int32),
            jax.ShapeDtypeStruct((1, 1), jnp.float32),
        ),
        interpret=_INTERPRET,
    )(x, Wg)


def _ffn_body(be_ref, xs_ref, w1_ref, b1_ref, w2_ref, b2_ref, y_ref,
              acc_ref, xsc_ref):
    j = pl.program_id(0)
    i = pl.program_id(1)
    bm = acc_ref.shape[0] // pl.num_programs(1)
    rows = pl.ds(i * bm, bm)

    @pl.when(j == 0)
    def _():
        xsc_ref[rows, :] = xs_ref[...].astype(jnp.bfloat16)

    xin = xsc_ref[rows, :].astype(jnp.float32)
    h = jax.lax.dot_general(
        xin, w1_ref[0], (((1,), (1,)), ((), ())),
        preferred_element_type=jnp.float32,
        precision=jax.lax.Precision.DEFAULT) + b1_ref[0]
    g = 0.5 * h * (1.0 + jax.lax.erf(h * (2.0 ** -0.5)))
    partial = jax.lax.dot_general(
        g, w2_ref[0], (((1,), (1,)), ((), ())),
        preferred_element_type=jnp.float32,
        precision=jax.lax.Precision.DEFAULT)

    @pl.when(j == 0)
    def _():
        acc_ref[rows, :] = partial.astype(jnp.bfloat16)

    @pl.when((j > 0) & (j < pl.num_programs(0) - 1))
    def _():
        acc_ref[rows, :] = (acc_ref[rows, :].astype(jnp.float32)
                            + partial).astype(jnp.bfloat16)

    @pl.when(j == pl.num_programs(0) - 1)
    def _():
        y_ref[...] = (acc_ref[rows, :].astype(jnp.float32) + partial
                      + b2_ref[0])


def _grouped_ffn(xs, W1, b1, W2, b2, block_expert, nb):
    npass = DFF // BDFF
    grid = (npass, nb)
    return pl.pallas_call(
        _ffn_body,
        grid_spec=pltpu.PrefetchScalarGridSpec(
            num_scalar_prefetch=1,
            grid=grid,
            in_specs=[
                pl.BlockSpec((BM, DIM),
                             lambda j, i, be: (jnp.where(j == 0, i, 0), 0)),
                pl.BlockSpec((1, BDFF, DIM), lambda j, i, be: (be[i], j, 0)),
                pl.BlockSpec((1, 1, BDFF), lambda j, i, be: (be[i], 0, j)),
                pl.BlockSpec((1, DIM, BDFF), lambda j, i, be: (be[i], 0, j)),
                pl.BlockSpec((1, 1, DIM), lambda j, i, be: (be[i], 0, 0)),
            ],
            out_specs=pl.BlockSpec(
                (BM, DIM),
                lambda j, i, be: (jnp.where(j == npass - 1, i, 0), 0)),
            scratch_shapes=[
                pltpu.VMEM((nb * BM, DIM), jnp.bfloat16),
                pltpu.VMEM((nb * BM, DIM), jnp.bfloat16),
            ],
        ),
        out_shape=jax.ShapeDtypeStruct((nb * BM, DIM), jnp.float32),
        interpret=_INTERPRET,
    )(block_expert, xs, W1, b1.reshape(E, 1, DFF), W2, b2.reshape(E, 1, DIM))


def _combine_body(x_ref, yp_ref, g_ref, o_ref):
    g0 = g_ref[:, 0:1]
    g1 = g_ref[:, 1:2]
    o_ref[...] = ((g0 + g1) * x_ref[...] + g0 * yp_ref[:, 0, :]
                  + g1 * yp_ref[:, 1, :])


def _combine(x, yp, gates):
    s = x.shape[0]
    bs = 512
    grid = (s // bs,)
    return pl.pallas_call(
        _combine_body,
        grid=grid,
        in_specs=[
            pl.BlockSpec((bs, DIM), lambda i: (i, 0)),
            pl.BlockSpec((bs, K, DIM), lambda i: (i, 0, 0)),
            pl.BlockSpec((bs, K), lambda i: (i, 0)),
        ],
        out_specs=pl.BlockSpec((bs, DIM), lambda i: (i, 0)),
        out_shape=jax.ShapeDtypeStruct((s, DIM), jnp.float32),
        interpret=_INTERPRET,
    )(x, yp, gates)


_SC_CHUNK = 32


def _sc_make_dispatch(s, p):
    n_pairs = s * K
    mesh = plsc.VectorSubcoreMesh(core_axis_name="c", subcore_axis_name="s")
    info = plsc.get_sparse_core_info()
    nw = info.num_cores * info.num_subcores
    per_w = n_pairs // nw
    nchunk = per_w // _SC_CHUNK

    @functools.partial(
        pl.kernel, mesh=mesh,
        out_type=jax.ShapeDtypeStruct((p, DIM), jnp.float32),
        scratch_types=[
            pltpu.VMEM((_SC_CHUNK,), jnp.int32),
            pltpu.VMEM((_SC_CHUNK,), jnp.int32),
            pltpu.VMEM((_SC_CHUNK, DIM), jnp.float32),
            pltpu.SemaphoreType.DMA,
        ],
    )
    def disp(x_hbm, tok_hbm, pos_hbm, xs_hbm, tok_v, idx_v, rows_v, sem):
        wid = lax.axis_index("s") * info.num_cores + lax.axis_index("c")
        for c in range(nchunk):
            base = wid * per_w + c * _SC_CHUNK
            pltpu.sync_copy(tok_hbm.at[pl.ds(base, _SC_CHUNK)], tok_v)
            pltpu.sync_copy(pos_hbm.at[pl.ds(base, _SC_CHUNK)], idx_v)
            pltpu.async_copy(x_hbm.at[tok_v], rows_v, sem).wait()
            pltpu.async_copy(rows_v, xs_hbm.at[idx_v], sem).wait()

    return disp


def _sc_make_gather(s, p):
    n_pairs = s * K
    mesh = plsc.VectorSubcoreMesh(core_axis_name="c", subcore_axis_name="s")
    info = plsc.get_sparse_core_info()
    nw = info.num_cores * info.num_subcores
    per_w = n_pairs // nw
    nchunk = per_w // _SC_CHUNK

    @functools.partial(
        pl.kernel, mesh=mesh,
        out_type=jax.ShapeDtypeStruct((n_pairs, DIM), jnp.float32),
        scratch_types=[
            pltpu.VMEM((_SC_CHUNK,), jnp.int32),
            pltpu.VMEM((_SC_CHUNK, DIM), jnp.float32),
            pltpu.SemaphoreType.DMA,
        ],
    )
    def gather(y_hbm, pos_hbm, yp_hbm, idx_v, rows_v, sem):
        wid = lax.axis_index("s") * info.num_cores + lax.axis_index("c")
        for c in range(nchunk):
            base = wid * per_w + c * _SC_CHUNK
            pltpu.sync_copy(pos_hbm.at[pl.ds(base, _SC_CHUNK)], idx_v)
            pltpu.async_copy(y_hbm.at[idx_v], rows_v, sem).wait()
            pltpu.sync_copy(rows_v, yp_hbm.at[pl.ds(base, _SC_CHUNK)])

    return gather


def kernel(x, Wg, W1, b1, W2, b2):
    s = x.shape[0]
    n_pairs = s * K
    nb = (n_pairs + E * BM) // BM

    gates, pos, block_expert, aux = _router(x, Wg, nb)
    pos_flat = pos.reshape(-1)
    tok = jnp.arange(n_pairs, dtype=jnp.int32) // K

    xs = _sc_make_dispatch(s, nb * BM)(x, tok, pos_flat)
    y = _grouped_ffn(xs, W1, b1, W2, b2, block_expert.reshape(nb), nb)

    yp = _sc_make_gather(s, nb * BM)(y, pos_flat)
    out = _combine(x, yp.reshape(s, K, DIM), gates)
    return out, aux.reshape(())


# speedup vs baseline: 1.3524x; 1.3524x over previous
import functools

import jax
import jax.numpy as jnp
from jax import lax
from jax.experimental import pallas as pl
from jax.experimental.pallas import tpu as pltpu
from jax.experimental.pallas import tpu_sc as plsc

DIM = 1024
DFF = 4096
E = 8
K = 2
BM = 256
BDFF = 1024

_INTERPRET = False


def _router_body(x_ref, wg_ref, vals_ref, pos_ref, be_ref, aux_ref):
    x = x_ref[...]
    s = x.shape[0]
    logits = jax.lax.dot_general(
        x, wg_ref[...], (((1,), (1,)), ((), ())),
        preferred_element_type=jnp.float32,
        precision=jax.lax.Precision.DEFAULT)
    m = jnp.max(logits, axis=-1, keepdims=True)
    ex = jnp.exp(logits - m)
    p = ex / jnp.sum(ex, axis=-1, keepdims=True)
    iota = jax.lax.broadcasted_iota(jnp.int32, p.shape, 1)
    v0 = jnp.max(p, axis=-1, keepdims=True)
    i0 = jnp.min(jnp.where(p == v0, iota, E), axis=-1, keepdims=True)
    oh0 = (iota == i0).astype(jnp.float32)
    p1 = jnp.where(oh0 > 0, -1.0, p)
    v1 = jnp.max(p1, axis=-1, keepdims=True)
    i1 = jnp.min(jnp.where(p1 == v1, iota, E), axis=-1, keepdims=True)
    oh1 = (iota == i1).astype(jnp.float32)
    vals_ref[...] = jnp.concatenate([v0, v1], axis=1)

    oh = oh0 + oh1
    cs = 256
    tri = (jax.lax.broadcasted_iota(jnp.int32, (cs, cs), 0)
           > jax.lax.broadcasted_iota(jnp.int32, (cs, cs), 1)
           ).astype(jnp.float32)
    chunks = []
    carry = jnp.zeros((1, E), jnp.float32)
    for c in range(s // cs):
        blk = oh[c * cs:(c + 1) * cs]
        local = jax.lax.dot_general(
            tri, blk, (((1,), (0,)), ((), ())),
            preferred_element_type=jnp.float32)
        chunks.append(local + carry)
        carry = carry + jnp.sum(blk, axis=0, keepdims=True)
    prev = jnp.concatenate(chunks, axis=0)
    counts = carry

    nblk = jnp.floor((counts + float(BM - 1)) * (1.0 / BM))
    tri8 = (jax.lax.broadcasted_iota(jnp.int32, (E, E), 0)
            <= jax.lax.broadcasted_iota(jnp.int32, (E, E), 1)
            ).astype(jnp.float32)
    ends = jax.lax.dot_general(
        nblk, tri8, (((1,), (0,)), ((), ())),
        preferred_element_type=jnp.float32)
    offs = (ends - nblk) * float(BM)
    rank0 = jnp.sum(prev * oh0, axis=1, keepdims=True)
    rank1 = jnp.sum(prev * oh1, axis=1, keepdims=True)
    off0 = jnp.sum(offs * oh0, axis=1, keepdims=True)
    off1 = jnp.sum(offs * oh1, axis=1, keepdims=True)
    pos_ref[...] = jnp.concatenate(
        [off0 + rank0, off1 + rank1], axis=1).astype(jnp.int32)

    nb = be_ref.shape[1]
    biota = jax.lax.broadcasted_iota(jnp.int32, (E, nb), 1).astype(jnp.float32)
    be = jnp.sum((biota >= ends.reshape(E, 1)).astype(jnp.int32),
                 axis=0, keepdims=True)
    be_ref[...] = jnp.minimum(be, E - 1)

    imp = jnp.sum(p, axis=0, keepdims=True)
    aux_ref[...] = (E / (float(s) * float(s))) * jnp.sum(
        imp * counts).reshape(1, 1)


def _router(x, Wg, nb):
    s = x.shape[0]
    return pl.pallas_call(
        _router_body,
        out_shape=(
            jax.ShapeDtypeStruct((s, K), jnp.float32),
            jax.ShapeDtypeStruct((s, K), jnp.int32),
            jax.ShapeDtypeStruct((1, nb), jnp.int32),
            jax.ShapeDtypeStruct((1, 1), jnp.float32),
        ),
        interpret=_INTERPRET,
    )(x, Wg)


def _ffn_body(be_ref, xs_ref, w1_ref, b1_ref, w2_ref, b2_ref, y_ref,
              acc_ref, xsc_ref):
    j = pl.program_id(0)
    i = pl.program_id(1)
    bm = acc_ref.shape[0] // pl.num_programs(1)
    rows = pl.ds(i * bm, bm)

    @pl.when(j == 0)
    def _():
        xsc_ref[rows, :] = xs_ref[...].astype(jnp.bfloat16)

    xin = xsc_ref[rows, :].astype(jnp.float32)
    h = jax.lax.dot_general(
        xin, w1_ref[0], (((1,), (1,)), ((), ())),
        preferred_element_type=jnp.float32,
        precision=jax.lax.Precision.DEFAULT) + b1_ref[0]
    g = 0.5 * h * (1.0 + jax.lax.erf(h * (2.0 ** -0.5)))
    partial = jax.lax.dot_general(
        g, w2_ref[0], (((1,), (1,)), ((), ())),
        preferred_element_type=jnp.float32,
        precision=jax.lax.Precision.DEFAULT)

    @pl.when(j == 0)
    def _():
        acc_ref[rows, :] = partial.astype(jnp.bfloat16)

    @pl.when((j > 0) & (j < pl.num_programs(0) - 1))
    def _():
        acc_ref[rows, :] = (acc_ref[rows, :].astype(jnp.float32)
                            + partial).astype(jnp.bfloat16)

    @pl.when(j == pl.num_programs(0) - 1)
    def _():
        y_ref[...] = (acc_ref[rows, :].astype(jnp.float32) + partial
                      + b2_ref[0])


def _grouped_ffn(xs, W1, b1, W2, b2, block_expert, nb):
    npass = DFF // BDFF
    grid = (npass, nb)
    return pl.pallas_call(
        _ffn_body,
        grid_spec=pltpu.PrefetchScalarGridSpec(
            num_scalar_prefetch=1,
            grid=grid,
            in_specs=[
                pl.BlockSpec((BM, DIM),
                             lambda j, i, be: (jnp.where(j == 0, i, 0), 0)),
                pl.BlockSpec((1, BDFF, DIM), lambda j, i, be: (be[i], j, 0)),
                pl.BlockSpec((1, 1, BDFF), lambda j, i, be: (be[i], 0, j)),
                pl.BlockSpec((1, DIM, BDFF), lambda j, i, be: (be[i], 0, j)),
                pl.BlockSpec((1, 1, DIM), lambda j, i, be: (be[i], 0, 0)),
            ],
            out_specs=pl.BlockSpec(
                (BM, DIM),
                lambda j, i, be: (jnp.where(j == npass - 1, i, 0), 0)),
            scratch_shapes=[
                pltpu.VMEM((nb * BM, DIM), jnp.bfloat16),
                pltpu.VMEM((nb * BM, DIM), jnp.bfloat16),
            ],
        ),
        out_shape=jax.ShapeDtypeStruct((nb * BM, DIM), jnp.float32),
        interpret=_INTERPRET,
    )(block_expert, xs, W1, b1.reshape(E, 1, DFF), W2, b2.reshape(E, 1, DIM))


def _combine_body(x_ref, yp_ref, g_ref, o_ref):
    g0 = g_ref[:, 0:1]
    g1 = g_ref[:, 1:2]
    o_ref[...] = ((g0 + g1) * x_ref[...] + g0 * yp_ref[:, 0, :]
                  + g1 * yp_ref[:, 1, :])


def _combine(x, yp, gates):
    s = x.shape[0]
    bs = 512
    grid = (s // bs,)
    return pl.pallas_call(
        _combine_body,
        grid=grid,
        in_specs=[
            pl.BlockSpec((bs, DIM), lambda i: (i, 0)),
            pl.BlockSpec((bs, K, DIM), lambda i: (i, 0, 0)),
            pl.BlockSpec((bs, K), lambda i: (i, 0)),
        ],
        out_specs=pl.BlockSpec((bs, DIM), lambda i: (i, 0)),
        out_shape=jax.ShapeDtypeStruct((s, DIM), jnp.float32),
        interpret=_INTERPRET,
    )(x, yp, gates)


_SC_CHUNK = 32


def _sc_make_dispatch(s, p):
    n_pairs = s * K
    mesh = plsc.VectorSubcoreMesh(core_axis_name="c", subcore_axis_name="s")
    info = plsc.get_sparse_core_info()
    nw = info.num_cores * info.num_subcores
    per_w = n_pairs // nw
    nchunk = per_w // _SC_CHUNK

    @functools.partial(
        pl.kernel, mesh=mesh,
        out_type=jax.ShapeDtypeStruct((p, DIM), jnp.float32),
        scratch_types=[
            pltpu.VMEM((_SC_CHUNK,), jnp.int32),
            pltpu.VMEM((_SC_CHUNK,), jnp.int32),
            pltpu.VMEM((_SC_CHUNK, DIM), jnp.float32),
            pltpu.SemaphoreType.DMA,
        ],
    )
    def disp(x_hbm, tok_hbm, pos_hbm, xs_hbm, tok_v, idx_v, rows_v, sem):
        wid = lax.axis_index("s") * info.num_cores + lax.axis_index("c")
        for c in range(nchunk):
            base = wid * per_w + c * _SC_CHUNK
            pltpu.sync_copy(tok_hbm.at[pl.ds(base, _SC_CHUNK)], tok_v)
            pltpu.sync_copy(pos_hbm.at[pl.ds(base, _SC_CHUNK)], idx_v)
            pltpu.async_copy(x_hbm.at[tok_v], rows_v, sem).wait()
            pltpu.async_copy(rows_v, xs_hbm.at[idx_v], sem).wait()

    return disp


def _sc_make_gather(s, p):
    n_pairs = s * K
    mesh = plsc.VectorSubcoreMesh(core_axis_name="c", subcore_axis_name="s")
    info = plsc.get_sparse_core_info()
    nw = info.num_cores * info.num_subcores
    per_w = n_pairs // nw
    nchunk = per_w // _SC_CHUNK

    @functools.partial(
        pl.kernel, mesh=mesh,
        out_type=jax.ShapeDtypeStruct((n_pairs, DIM), jnp.float32),
        scratch_types=[
            pltpu.VMEM((_SC_CHUNK,), jnp.int32),
            pltpu.VMEM((_SC_CHUNK, DIM), jnp.float32),
            pltpu.SemaphoreType.DMA,
        ],
    )
    def gather(y_hbm, pos_hbm, yp_hbm, idx_v, rows_v, sem):
        wid = lax.axis_index("s") * info.num_cores + lax.axis_index("c")
        for c in range(nchunk):
            base = wid * per_w + c * _SC_CHUNK
            pltpu.sync_copy(pos_hbm.at[pl.ds(base, _SC_CHUNK)], idx_v)
            pltpu.async_copy(y_hbm.at[idx_v], rows_v, sem).wait()
            pltpu.sync_copy(rows_v, yp_hbm.at[pl.ds(base, _SC_CHUNK)])

    return gather


def kernel(x, Wg, W1, b1, W2, b2):
    s = x.shape[0]
    n_pairs = s * K
    nb = (n_pairs + E * BM) // BM

    gates, pos, block_expert, aux = _router(x, Wg, nb)
    pos_flat = pos.reshape(-1)
    tok = jnp.arange(n_pairs, dtype=jnp.int32) // K

    xs = _sc_make_dispatch(s, nb * BM)(x, tok, pos_flat)
    y = _grouped_ffn(xs, W1, b1, W2, b2, block_expert.reshape(nb), nb)

    yp = _sc_make_gather(s, nb * BM)(y, pos_flat)
    out = _combine(x, yp.reshape(s, K, DIM), gates)
    return out, aux.reshape(())


# speedup vs baseline: 1.4960x; 1.1061x over previous
import functools

import jax
import jax.numpy as jnp
from jax import lax
from jax.experimental import pallas as pl
from jax.experimental.pallas import tpu as pltpu
from jax.experimental.pallas import tpu_sc as plsc

DIM = 1024
DFF = 4096
E = 8
K = 2
BM = 256
BDFF = 2048

_INTERPRET = False


def _router_body(x_ref, wg_ref, vals_ref, pos_ref, be_ref, aux_ref):
    x = x_ref[...]
    s = x.shape[0]
    logits = jax.lax.dot_general(
        x, wg_ref[...], (((1,), (1,)), ((), ())),
        preferred_element_type=jnp.float32,
        precision=jax.lax.Precision.DEFAULT)
    m = jnp.max(logits, axis=-1, keepdims=True)
    ex = jnp.exp(logits - m)
    p = ex / jnp.sum(ex, axis=-1, keepdims=True)
    iota = jax.lax.broadcasted_iota(jnp.int32, p.shape, 1)
    v0 = jnp.max(p, axis=-1, keepdims=True)
    i0 = jnp.min(jnp.where(p == v0, iota, E), axis=-1, keepdims=True)
    oh0 = (iota == i0).astype(jnp.float32)
    p1 = jnp.where(oh0 > 0, -1.0, p)
    v1 = jnp.max(p1, axis=-1, keepdims=True)
    i1 = jnp.min(jnp.where(p1 == v1, iota, E), axis=-1, keepdims=True)
    oh1 = (iota == i1).astype(jnp.float32)
    vals_ref[...] = jnp.concatenate([v0, v1], axis=1)

    oh = oh0 + oh1
    cs = 256
    tri = (jax.lax.broadcasted_iota(jnp.int32, (cs, cs), 0)
           > jax.lax.broadcasted_iota(jnp.int32, (cs, cs), 1)
           ).astype(jnp.float32)
    chunks = []
    carry = jnp.zeros((1, E), jnp.float32)
    for c in range(s // cs):
        blk = oh[c * cs:(c + 1) * cs]
        local = jax.lax.dot_general(
            tri, blk, (((1,), (0,)), ((), ())),
            preferred_element_type=jnp.float32)
        chunks.append(local + carry)
        carry = carry + jnp.sum(blk, axis=0, keepdims=True)
    prev = jnp.concatenate(chunks, axis=0)
    counts = carry

    nblk = jnp.floor((counts + float(BM - 1)) * (1.0 / BM))
    tri8 = (jax.lax.broadcasted_iota(jnp.int32, (E, E), 0)
            <= jax.lax.broadcasted_iota(jnp.int32, (E, E), 1)
            ).astype(jnp.float32)
    ends = jax.lax.dot_general(
        nblk, tri8, (((1,), (0,)), ((), ())),
        preferred_element_type=jnp.float32)
    offs = (ends - nblk) * float(BM)
    rank0 = jnp.sum(prev * oh0, axis=1, keepdims=True)
    rank1 = jnp.sum(prev * oh1, axis=1, keepdims=True)
    off0 = jnp.sum(offs * oh0, axis=1, keepdims=True)
    off1 = jnp.sum(offs * oh1, axis=1, keepdims=True)
    pos_ref[...] = jnp.concatenate(
        [off0 + rank0, off1 + rank1], axis=1).astype(jnp.int32)

    nb = be_ref.shape[1]
    biota = jax.lax.broadcasted_iota(jnp.int32, (E, nb), 1).astype(jnp.float32)
    be = jnp.sum((biota >= ends.reshape(E, 1)).astype(jnp.int32),
                 axis=0, keepdims=True)
    be_ref[...] = jnp.minimum(be, E - 1)

    imp = jnp.sum(p, axis=0, keepdims=True)
    aux_ref[...] = (E / (float(s) * float(s))) * jnp.sum(
        imp * counts).reshape(1, 1)


def _router(x, Wg, nb):
    s = x.shape[0]
    return pl.pallas_call(
        _router_body,
        out_shape=(
            jax.ShapeDtypeStruct((s, K), jnp.float32),
            jax.ShapeDtypeStruct((s, K), jnp.int32),
            jax.ShapeDtypeStruct((1, nb), jnp.int32),
            jax.ShapeDtypeStruct((1, 1), jnp.float32),
        ),
        interpret=_INTERPRET,
    )(x, Wg)


def _ffn_body(be_ref, xs_ref, w1_ref, b1_ref, w2_ref, b2_ref, y_ref,
              acc_ref):
    j = pl.program_id(0)
    i = pl.program_id(1)
    bm = acc_ref.shape[0] // pl.num_programs(1)
    rows = pl.ds(i * bm, bm)

    xin = xs_ref[...]
    h = jax.lax.dot_general(
        xin, w1_ref[0], (((1,), (1,)), ((), ())),
        preferred_element_type=jnp.float32,
        precision=jax.lax.Precision.DEFAULT) + b1_ref[0]
    g = 0.5 * h * (1.0 + jax.lax.erf(h * (2.0 ** -0.5)))
    partial = jax.lax.dot_general(
        g, w2_ref[0], (((1,), (1,)), ((), ())),
        preferred_element_type=jnp.float32,
        precision=jax.lax.Precision.DEFAULT)

    @pl.when(j == 0)
    def _():
        acc_ref[rows, :] = partial.astype(jnp.bfloat16)

    @pl.when((j > 0) & (j < pl.num_programs(0) - 1))
    def _():
        acc_ref[rows, :] = (acc_ref[rows, :].astype(jnp.float32)
                            + partial).astype(jnp.bfloat16)

    @pl.when(j == pl.num_programs(0) - 1)
    def _():
        y_ref[...] = (acc_ref[rows, :].astype(jnp.float32) + partial
                      + b2_ref[0])


def _grouped_ffn(xs, W1, b1, W2, b2, block_expert, nb):
    npass = DFF // BDFF
    grid = (npass, nb)
    return pl.pallas_call(
        _ffn_body,
        grid_spec=pltpu.PrefetchScalarGridSpec(
            num_scalar_prefetch=1,
            grid=grid,
            in_specs=[
                pl.BlockSpec((BM, DIM), lambda j, i, be: (i, 0)),
                pl.BlockSpec((1, BDFF, DIM), lambda j, i, be: (be[i], j, 0)),
                pl.BlockSpec((1, 1, BDFF), lambda j, i, be: (be[i], 0, j)),
                pl.BlockSpec((1, DIM, BDFF), lambda j, i, be: (be[i], 0, j)),
                pl.BlockSpec((1, 1, DIM), lambda j, i, be: (be[i], 0, 0)),
            ],
            out_specs=pl.BlockSpec(
                (BM, DIM),
                lambda j, i, be: (jnp.where(j == npass - 1, i, 0), 0)),
            scratch_shapes=[
                pltpu.VMEM((nb * BM, DIM), jnp.bfloat16),
            ],
        ),
        out_shape=jax.ShapeDtypeStruct((nb * BM, DIM), jnp.float32),
        interpret=_INTERPRET,
    )(block_expert, xs, W1, b1.reshape(E, 1, DFF), W2, b2.reshape(E, 1, DIM))


def _combine_body(x_ref, yp_ref, g_ref, o_ref):
    g0 = g_ref[:, 0:1]
    g1 = g_ref[:, 1:2]
    o_ref[...] = ((g0 + g1) * x_ref[...] + g0 * yp_ref[:, 0, :]
                  + g1 * yp_ref[:, 1, :])


def _combine(x, yp, gates):
    s = x.shape[0]
    bs = 512
    grid = (s // bs,)
    return pl.pallas_call(
        _combine_body,
        grid=grid,
        in_specs=[
            pl.BlockSpec((bs, DIM), lambda i: (i, 0)),
            pl.BlockSpec((bs, K, DIM), lambda i: (i, 0, 0)),
            pl.BlockSpec((bs, K), lambda i: (i, 0)),
        ],
        out_specs=pl.BlockSpec((bs, DIM), lambda i: (i, 0)),
        out_shape=jax.ShapeDtypeStruct((s, DIM), jnp.float32),
        interpret=_INTERPRET,
    )(x, yp, gates)


_SC_CHUNK = 32


def _sc_make_dispatch(s, p):
    n_pairs = s * K
    mesh = plsc.VectorSubcoreMesh(core_axis_name="c", subcore_axis_name="s")
    info = plsc.get_sparse_core_info()
    nw = info.num_cores * info.num_subcores
    per_w = n_pairs // nw
    nchunk = per_w // _SC_CHUNK

    @functools.partial(
        pl.kernel, mesh=mesh,
        out_type=jax.ShapeDtypeStruct((p, DIM), jnp.float32),
        scratch_types=[
            pltpu.VMEM((_SC_CHUNK,), jnp.int32),
            pltpu.VMEM((_SC_CHUNK,), jnp.int32),
            pltpu.VMEM((_SC_CHUNK, DIM), jnp.float32),
            pltpu.SemaphoreType.DMA,
        ],
    )
    def disp(x_hbm, tok_hbm, pos_hbm, xs_hbm, tok_v, idx_v, rows_v, sem):
        wid = lax.axis_index("s") * info.num_cores + lax.axis_index("c")
        for c in range(nchunk):
            base = wid * per_w + c * _SC_CHUNK
            pltpu.sync_copy(tok_hbm.at[pl.ds(base, _SC_CHUNK)], tok_v)
            pltpu.sync_copy(pos_hbm.at[pl.ds(base, _SC_CHUNK)], idx_v)
            pltpu.async_copy(x_hbm.at[tok_v], rows_v, sem).wait()
            pltpu.async_copy(rows_v, xs_hbm.at[idx_v], sem).wait()

    return disp


def _sc_make_gather(s, p):
    n_pairs = s * K
    mesh = plsc.VectorSubcoreMesh(core_axis_name="c", subcore_axis_name="s")
    info = plsc.get_sparse_core_info()
    nw = info.num_cores * info.num_subcores
    per_w = n_pairs // nw
    nchunk = per_w // _SC_CHUNK

    @functools.partial(
        pl.kernel, mesh=mesh,
        out_type=jax.ShapeDtypeStruct((n_pairs, DIM), jnp.float32),
        scratch_types=[
            pltpu.VMEM((_SC_CHUNK,), jnp.int32),
            pltpu.VMEM((_SC_CHUNK, DIM), jnp.float32),
            pltpu.SemaphoreType.DMA,
        ],
    )
    def gather(y_hbm, pos_hbm, yp_hbm, idx_v, rows_v, sem):
        wid = lax.axis_index("s") * info.num_cores + lax.axis_index("c")
        for c in range(nchunk):
            base = wid * per_w + c * _SC_CHUNK
            pltpu.sync_copy(pos_hbm.at[pl.ds(base, _SC_CHUNK)], idx_v)
            pltpu.async_copy(y_hbm.at[idx_v], rows_v, sem).wait()
            pltpu.sync_copy(rows_v, yp_hbm.at[pl.ds(base, _SC_CHUNK)])

    return gather


def kernel(x, Wg, W1, b1, W2, b2):
    s = x.shape[0]
    n_pairs = s * K
    nb = (n_pairs + E * BM) // BM

    gates, pos, block_expert, aux = _router(x, Wg, nb)
    pos_flat = pos.reshape(-1)
    tok = jnp.arange(n_pairs, dtype=jnp.int32) // K

    xs = _sc_make_dispatch(s, nb * BM)(x, tok, pos_flat)
    y = _grouped_ffn(xs, W1, b1, W2, b2, block_expert.reshape(nb), nb)

    yp = _sc_make_gather(s, nb * BM)(y, pos_flat)
    out = _combine(x, yp.reshape(s, K, DIM), gates)
    return out, aux.reshape(())
